# Initial kernel scaffold; baseline (speedup 1.0000x reference)
import jax
import jax.numpy as jnp
from jax.experimental import pallas as pl


def kernel(x, edge_index, W1, b1, W2, b2, Wp, bp):
    raise NotImplementedError("write your pallas kernel here")


# speedup vs baseline: 7.5927x; 7.5927x over previous
import functools

import jax
import jax.numpy as jnp
from jax import lax
from jax.experimental import pallas as pl
from jax.experimental.pallas import tpu as pltpu
from jax.experimental.pallas import tpu_sc as plsc

N_NODES = 10000
D = 128

NC = 2
NS = 16
NW = NC * NS

N_PAD = 10240
ROWS_PER_TILE = N_PAD // NS
K = 128
E_PAD_PER_W = 10240
NCH = E_PAD_PER_W // K
E_PAD = NW * E_PAD_PER_W

_mesh = lambda: plsc.VectorSubcoreMesh(
    core_axis_name="c", subcore_axis_name="s", num_cores=NC, num_subcores=NS
)


def _zero_rows(buf, nrows, width):
    z16 = jnp.zeros((16,), jnp.float32)

    def body(i, _):
        for j in range(width // 16):
            buf[i, pl.ds(j * 16, 16)] = z16
        return 0

    lax.fori_loop(0, nrows, body, 0)


def _fill_ones(buf, nrows, width):
    o16 = jnp.ones((16,), jnp.float32)

    def body(i, _):
        for j in range(width // 16):
            buf[i, pl.ds(j * 16, 16)] = o16
        return 0

    lax.fori_loop(0, nrows, body, 0)


@functools.cache
def _sc_deg_hist_kernel():
    return functools.partial(
        pl.kernel,
        out_type=jax.ShapeDtypeStruct((NC, N_PAD, D), jnp.float32),
        mesh=_mesh(),
        scratch_types=[
            pltpu.VMEM((NCH, K), jnp.int32),
            pltpu.VMEM((K, D), jnp.float32),
            pltpu.VMEM_SHARED((N_PAD, D), jnp.float32),
        ],
        name="sc_deg_hist",
    )(_sc_deg_hist_body)


def _sc_deg_hist_body(dst_hbm, out_hbm, dst_v, buf_v, acc_sh):
    c = lax.axis_index("c")
    s = lax.axis_index("s")
    wid = c * NS + s
    base = s * ROWS_PER_TILE

    _zero_rows(buf_v, K, D)
    for r in range(ROWS_PER_TILE // K):
        pltpu.sync_copy(buf_v, acc_sh.at[pl.ds(base + r * K, K)])
    plsc.subcore_barrier()

    _fill_ones(buf_v, K, D)
    pltpu.sync_copy(dst_hbm.at[wid], dst_v)

    def chunk(j, _):
        pltpu.sync_copy(buf_v, acc_sh.at[dst_v.at[j]], add=True)
        return 0

    lax.fori_loop(0, NCH, chunk, 0)
    plsc.subcore_barrier()

    pltpu.sync_copy(
        acc_sh.at[pl.ds(base, ROWS_PER_TILE)],
        out_hbm.at[c, pl.ds(base, ROWS_PER_TILE)],
    )


@functools.cache
def _sc_seg_sum_kernel():
    return functools.partial(
        pl.kernel,
        out_type=jax.ShapeDtypeStruct((NC, N_PAD, D), jnp.float32),
        mesh=_mesh(),
        scratch_types=[
            pltpu.VMEM((NCH, K), jnp.int32),
            pltpu.VMEM((NCH, K), jnp.int32),
            pltpu.VMEM((K, D), jnp.float32),
            pltpu.VMEM_SHARED((N_PAD, D), jnp.float32),
            pltpu.SemaphoreType.DMA,
        ],
        name="sc_seg_sum",
    )(_sc_seg_sum_body)


def _sc_seg_sum_body(src_hbm, dst_hbm, g_hbm, out_hbm, src_v, dst_v, rows_v, acc_sh, sem):
    c = lax.axis_index("c")
    s = lax.axis_index("s")
    wid = c * NS + s
    base = s * ROWS_PER_TILE

    _zero_rows(rows_v, K, D)
    for r in range(ROWS_PER_TILE // K):
        pltpu.sync_copy(rows_v, acc_sh.at[pl.ds(base + r * K, K)])
    plsc.subcore_barrier()

    pltpu.sync_copy(src_hbm.at[wid], src_v)
    pltpu.sync_copy(dst_hbm.at[wid], dst_v)

    def chunk(j, _):
        pltpu.async_copy(g_hbm.at[src_v.at[j]], rows_v, sem).wait()
        pltpu.sync_copy(rows_v, acc_sh.at[dst_v.at[j]], add=True)
        return 0

    lax.fori_loop(0, NCH, chunk, 0)
    plsc.subcore_barrier()

    pltpu.sync_copy(
        acc_sh.at[pl.ds(base, ROWS_PER_TILE)],
        out_hbm.at[c, pl.ds(base, ROWS_PER_TILE)],
    )


def _dinv_block(hist_blk):
    deg = hist_blk[0, :, :1] + hist_blk[1, :, :1] + 1.0
    return lax.rsqrt(deg)


def _tc_a_body(hist_ref, x_ref, w1_ref, g1_ref):
    dinv = _dinv_block(hist_ref[...])
    h = jnp.dot(x_ref[...], w1_ref[...], preferred_element_type=jnp.float32)
    g1_ref[...] = dinv * h


def _tc_b_body(hist_ref, agg_ref, g1_ref, b1_ref, w2_ref, g2_ref):
    dinv = _dinv_block(hist_ref[...])
    a = agg_ref[0] + agg_ref[1] + g1_ref[...]
    u = jnp.maximum(dinv * a + b1_ref[...], 0.0)
    g2_ref[...] = dinv * jnp.dot(u, w2_ref[...], preferred_element_type=jnp.float32)


def _tc_c_body(hist_ref, agg_ref, g2_ref, b2_ref, wp_ref, bp_ref, out_ref):
    dinv = _dinv_block(hist_ref[...])
    a = agg_ref[0] + agg_ref[1] + g2_ref[...]
    o = dinv * a + b2_ref[...]
    out_ref[...] = jnp.dot(o, wp_ref[...], preferred_element_type=jnp.float32) + bp_ref[...]


def _row_specs(rb, extra):
    hist = pl.BlockSpec((2, rb, D), lambda i: (0, i, 0))
    row = pl.BlockSpec((rb, D), lambda i: (i, 0))
    agg = pl.BlockSpec((2, rb, D), lambda i: (0, i, 0))
    full = pl.BlockSpec((1, D), lambda i: (0, 0))
    mat = pl.BlockSpec((D, D), lambda i: (0, 0))
    m = {"hist": hist, "row": row, "agg": agg, "full": full, "mat": mat}
    return [m[e] for e in extra]


def _tc_a(hist, x_pad, w1):
    rb = N_PAD // 8
    return pl.pallas_call(
        _tc_a_body,
        grid=(8,),
        in_specs=_row_specs(rb, ["hist", "row", "mat"]),
        out_specs=pl.BlockSpec((rb, D), lambda i: (i, 0)),
        out_shape=jax.ShapeDtypeStruct((N_PAD, D), jnp.float32),
    )(hist, x_pad, w1)


def _tc_b(hist, agg, g1, b1, w2):
    rb = N_PAD // 8
    return pl.pallas_call(
        _tc_b_body,
        grid=(8,),
        in_specs=_row_specs(rb, ["hist", "agg", "row", "full", "mat"]),
        out_specs=pl.BlockSpec((rb, D), lambda i: (i, 0)),
        out_shape=jax.ShapeDtypeStruct((N_PAD, D), jnp.float32),
    )(hist, agg, g1, b1, w2)


def _tc_c(hist, agg, g2, b2, wp, bp):
    rb = 1256
    return pl.pallas_call(
        _tc_c_body,
        grid=(8,),
        in_specs=_row_specs(rb, ["hist", "agg", "row", "full", "mat", "full"]),
        out_specs=pl.BlockSpec((rb, D), lambda i: (i, 0)),
        out_shape=jax.ShapeDtypeStruct((N_NODES, D), jnp.float32),
    )(hist, agg, g2, b2, wp, bp)


def kernel(x, edge_index, W1, b1, W2, b2, Wp, bp):
    src = edge_index[0].astype(jnp.int32)
    dst = edge_index[1].astype(jnp.int32)

    fill = jnp.full((E_PAD,), N_NODES, jnp.int32)
    src3 = fill.at[: src.shape[0]].set(src).reshape(NW, NCH, K)
    dst3 = fill.at[: dst.shape[0]].set(dst).reshape(NW, NCH, K)

    x_pad = jnp.zeros((N_PAD, D), jnp.float32).at[:N_NODES].set(x)
    b1r = b1.reshape(1, D)
    b2r = b2.reshape(1, D)
    bpr = bp.reshape(1, D)

    hist = _sc_deg_hist_kernel()(dst3)
    g1 = _tc_a(hist, x_pad, W1)
    agg1 = _sc_seg_sum_kernel()(src3, dst3, g1)
    g2 = _tc_b(hist, agg1, g1, b1r, W2)
    agg2 = _sc_seg_sum_kernel()(src3, dst3, g2)
    out = _tc_c(hist, agg2, g2, b2r, Wp, bpr)
    return out


# speedup vs baseline: 8.4209x; 1.1091x over previous
import functools

import jax
import jax.numpy as jnp
from jax import lax
from jax.experimental import pallas as pl
from jax.experimental.pallas import tpu as pltpu
from jax.experimental.pallas import tpu_sc as plsc

N_NODES = 10000
D = 128

NC = 2
NS = 16
NW = NC * NS

N_PAD = 10240
ROWS_PER_TILE = N_PAD // NS
K = 128
E_PAD_PER_W = 10240
NCH = E_PAD_PER_W // K
E_PAD = NW * E_PAD_PER_W
NBUF = 2
GS = 8

_mesh = lambda: plsc.VectorSubcoreMesh(
    core_axis_name="c", subcore_axis_name="s", num_cores=NC, num_subcores=NS
)


def _zero_rows(buf, nrows, width):
    z16 = jnp.zeros((16,), jnp.float32)

    def body(i, _):
        for j in range(width // 16):
            buf[i, pl.ds(j * 16, 16)] = z16
        return 0

    lax.fori_loop(0, nrows, body, 0)


def _zero_rows3(buf):
    z16 = jnp.zeros((16,), jnp.float32)

    def body(i, _):
        for j in range(D // 16):
            buf[0, i, pl.ds(j * 16, 16)] = z16
        return 0

    lax.fori_loop(0, K, body, 0)


def _fill_ones(buf, nrows, width):
    o16 = jnp.ones((16,), jnp.float32)

    def body(i, _):
        for j in range(width // 16):
            buf[i, pl.ds(j * 16, 16)] = o16
        return 0

    lax.fori_loop(0, nrows, body, 0)


@functools.cache
def _sc_deg_hist_kernel():
    return functools.partial(
        pl.kernel,
        out_type=jax.ShapeDtypeStruct((NC, N_PAD, D), jnp.float32),
        mesh=_mesh(),
        scratch_types=[
            pltpu.VMEM((NCH, K), jnp.int32),
            pltpu.VMEM((K, D), jnp.float32),
            pltpu.VMEM_SHARED((N_PAD, D), jnp.float32),
        ],
        name="sc_deg_hist",
    )(_sc_deg_hist_body)


def _sc_deg_hist_body(dst_hbm, out_hbm, dst_v, buf_v, acc_sh):
    c = lax.axis_index("c")
    s = lax.axis_index("s")
    wid = c * NS + s
    base = s * ROWS_PER_TILE

    _zero_rows(buf_v, K, D)
    for r in range(ROWS_PER_TILE // K):
        pltpu.sync_copy(buf_v, acc_sh.at[pl.ds(base + r * K, K)])
    plsc.subcore_barrier()

    _fill_ones(buf_v, K, D)
    pltpu.sync_copy(dst_hbm.at[wid], dst_v)

    def chunk(j, _):
        pltpu.sync_copy(buf_v, acc_sh.at[dst_v.at[j]], add=True)
        return 0

    lax.fori_loop(0, NCH, chunk, 0)
    plsc.subcore_barrier()

    pltpu.sync_copy(
        acc_sh.at[pl.ds(base, ROWS_PER_TILE)],
        out_hbm.at[c, pl.ds(base, ROWS_PER_TILE)],
    )


@functools.cache
def _sc_seg_sum_kernel():
    return functools.partial(
        pl.kernel,
        out_type=jax.ShapeDtypeStruct((NC, N_PAD, D), jnp.float32),
        mesh=_mesh(),
        scratch_types=[
            pltpu.VMEM((NCH, K), jnp.int32),
            pltpu.VMEM((GS, K), jnp.int32),
            pltpu.VMEM((NBUF, K, D), jnp.float32),
            pltpu.VMEM_SHARED((N_PAD, D), jnp.float32),
            [pltpu.SemaphoreType.DMA] * NBUF,
        ],
        name="sc_seg_sum",
    )(_sc_seg_sum_body)


def _sc_seg_sum_body(
    src_hbm, dst_hbm, g_hbm, out_hbm, src_v, dst_sl, rows_v, acc_sh, gsem
):
    c = lax.axis_index("c")
    s = lax.axis_index("s")
    wid = c * NS + s
    base = s * ROWS_PER_TILE

    _zero_rows3(rows_v)
    for r in range(ROWS_PER_TILE // K):
        pltpu.sync_copy(rows_v.at[0], acc_sh.at[pl.ds(base + r * K, K)])
    plsc.subcore_barrier()

    pltpu.sync_copy(src_hbm.at[wid], src_v)

    def gather(j, b):
        pltpu.async_copy(g_hbm.at[src_v.at[j]], rows_v.at[b], gsem[b])

    def gather_wait(j, b):
        pltpu.make_async_copy(g_hbm.at[src_v.at[j]], rows_v.at[b], gsem[b]).wait()

    for b in range(NBUF):
        gather(jnp.int32(b), b)

    def slab(g, _):
        pltpu.sync_copy(dst_hbm.at[wid, pl.ds(g * GS, GS)], dst_sl)
        for b8 in range(GS):
            j = g * GS + b8
            b = b8 % NBUF
            gather_wait(j, b)
            pltpu.sync_copy(rows_v.at[b], acc_sh.at[dst_sl.at[b8]], add=True)
            jn = j + NBUF

            @pl.when(jn < NCH)
            def _():
                gather(jn, b)

        return 0

    lax.fori_loop(0, NCH // GS, slab, 0)
    plsc.subcore_barrier()

    pltpu.sync_copy(
        acc_sh.at[pl.ds(base, ROWS_PER_TILE)],
        out_hbm.at[c, pl.ds(base, ROWS_PER_TILE)],
    )


def _dinv_block(hist_blk):
    deg = hist_blk[0, :, :1] + hist_blk[1, :, :1] + 1.0
    return lax.rsqrt(deg)


def _tc_a_body(hist_ref, x_ref, w1_ref, g1_ref):
    dinv = _dinv_block(hist_ref[...])
    h = jnp.dot(x_ref[...], w1_ref[...], preferred_element_type=jnp.float32)
    g1_ref[...] = dinv * h


def _tc_b_body(hist_ref, agg_ref, g1_ref, b1_ref, w2_ref, g2_ref):
    dinv = _dinv_block(hist_ref[...])
    a = agg_ref[0] + agg_ref[1] + g1_ref[...]
    u = jnp.maximum(dinv * a + b1_ref[...], 0.0)
    g2_ref[...] = dinv * jnp.dot(u, w2_ref[...], preferred_element_type=jnp.float32)


def _tc_c_body(hist_ref, agg_ref, g2_ref, b2_ref, wp_ref, bp_ref, out_ref):
    dinv = _dinv_block(hist_ref[...])
    a = agg_ref[0] + agg_ref[1] + g2_ref[...]
    o = dinv * a + b2_ref[...]
    out_ref[...] = jnp.dot(o, wp_ref[...], preferred_element_type=jnp.float32) + bp_ref[...]


def _row_specs(rb, extra):
    hist = pl.BlockSpec((2, rb, D), lambda i: (0, i, 0))
    row = pl.BlockSpec((rb, D), lambda i: (i, 0))
    agg = pl.BlockSpec((2, rb, D), lambda i: (0, i, 0))
    full = pl.BlockSpec((1, D), lambda i: (0, 0))
    mat = pl.BlockSpec((D, D), lambda i: (0, 0))
    m = {"hist": hist, "row": row, "agg": agg, "full": full, "mat": mat}
    return [m[e] for e in extra]


def _tc_a(hist, x_pad, w1):
    rb = N_PAD // 8
    return pl.pallas_call(
        _tc_a_body,
        grid=(8,),
        in_specs=_row_specs(rb, ["hist", "row", "mat"]),
        out_specs=pl.BlockSpec((rb, D), lambda i: (i, 0)),
        out_shape=jax.ShapeDtypeStruct((N_PAD, D), jnp.float32),
    )(hist, x_pad, w1)


def _tc_b(hist, agg, g1, b1, w2):
    rb = N_PAD // 8
    return pl.pallas_call(
        _tc_b_body,
        grid=(8,),
        in_specs=_row_specs(rb, ["hist", "agg", "row", "full", "mat"]),
        out_specs=pl.BlockSpec((rb, D), lambda i: (i, 0)),
        out_shape=jax.ShapeDtypeStruct((N_PAD, D), jnp.float32),
    )(hist, agg, g1, b1, w2)


def _tc_c(hist, agg, g2, b2, wp, bp):
    rb = 1256
    return pl.pallas_call(
        _tc_c_body,
        grid=(8,),
        in_specs=_row_specs(rb, ["hist", "agg", "row", "full", "mat", "full"]),
        out_specs=pl.BlockSpec((rb, D), lambda i: (i, 0)),
        out_shape=jax.ShapeDtypeStruct((N_NODES, D), jnp.float32),
    )(hist, agg, g2, b2, wp, bp)


def kernel(x, edge_index, W1, b1, W2, b2, Wp, bp):
    src = edge_index[0].astype(jnp.int32)
    dst = edge_index[1].astype(jnp.int32)

    fill = jnp.full((E_PAD,), N_NODES, jnp.int32)
    src3 = fill.at[: src.shape[0]].set(src).reshape(NW, NCH, K)
    dst3 = fill.at[: dst.shape[0]].set(dst).reshape(NW, NCH, K)

    x_pad = jnp.zeros((N_PAD, D), jnp.float32).at[:N_NODES].set(x)
    b1r = b1.reshape(1, D)
    b2r = b2.reshape(1, D)
    bpr = bp.reshape(1, D)

    hist = _sc_deg_hist_kernel()(dst3)
    g1 = _tc_a(hist, x_pad, W1)
    agg1 = _sc_seg_sum_kernel()(src3, dst3, g1)
    g2 = _tc_b(hist, agg1, g1, b1r, W2)
    agg2 = _sc_seg_sum_kernel()(src3, dst3, g2)
    out = _tc_c(hist, agg2, g2, b2r, Wp, bpr)
    return out


# speedup vs baseline: 9.8438x; 1.1690x over previous
import functools

import jax
import jax.numpy as jnp
from jax import lax
from jax.experimental import pallas as pl
from jax.experimental.pallas import tpu as pltpu
from jax.experimental.pallas import tpu_sc as plsc

N_NODES = 10000
D = 128

NC = 2
NS = 16
NW = NC * NS

N_PAD = 10240
ROWS_PER_TILE = N_PAD // NS
K = 128
E_PAD_PER_W = 10240
NCH = E_PAD_PER_W // K
E_PAD = NW * E_PAD_PER_W
NBUF = 2
GS = 8
NCH_C = (32, 128)
CH_TOT = E_PAD // K

_mesh = lambda: plsc.VectorSubcoreMesh(
    core_axis_name="c", subcore_axis_name="s", num_cores=NC, num_subcores=NS
)


def _zero_rows(buf, nrows, width):
    z16 = jnp.zeros((16,), jnp.float32)

    def body(i, _):
        for j in range(width // 16):
            buf[i, pl.ds(j * 16, 16)] = z16
        return 0

    lax.fori_loop(0, nrows, body, 0)


def _zero_rows3(buf):
    z16 = jnp.zeros((16,), jnp.float32)

    def body(i, _):
        for j in range(D // 16):
            buf[0, i, pl.ds(j * 16, 16)] = z16
        return 0

    lax.fori_loop(0, K, body, 0)


def _fill_ones(buf, nrows, width):
    o16 = jnp.ones((16,), jnp.float32)

    def body(i, _):
        for j in range(width // 16):
            buf[i, pl.ds(j * 16, 16)] = o16
        return 0

    lax.fori_loop(0, nrows, body, 0)


@functools.cache
def _sc_deg_hist_kernel():
    return functools.partial(
        pl.kernel,
        out_type=jax.ShapeDtypeStruct((NC, N_PAD, D), jnp.float32),
        mesh=_mesh(),
        scratch_types=[
            pltpu.VMEM((NCH, K), jnp.int32),
            pltpu.VMEM((K, D), jnp.float32),
            pltpu.VMEM_SHARED((N_PAD, D), jnp.float32),
        ],
        name="sc_deg_hist",
    )(_sc_deg_hist_body)


def _sc_deg_hist_body(dst_hbm, out_hbm, dst_v, buf_v, acc_sh):
    c = lax.axis_index("c")
    s = lax.axis_index("s")
    wid = c * NS + s
    base = s * ROWS_PER_TILE

    _zero_rows(buf_v, K, D)
    for r in range(ROWS_PER_TILE // K):
        pltpu.sync_copy(buf_v, acc_sh.at[pl.ds(base + r * K, K)])
    plsc.subcore_barrier()

    _fill_ones(buf_v, K, D)
    pltpu.sync_copy(dst_hbm.at[wid], dst_v)

    def chunk(j, _):
        pltpu.sync_copy(buf_v, acc_sh.at[dst_v.at[j]], add=True)
        return 0

    lax.fori_loop(0, NCH, chunk, 0)
    plsc.subcore_barrier()

    pltpu.sync_copy(
        acc_sh.at[pl.ds(base, ROWS_PER_TILE)],
        out_hbm.at[c, pl.ds(base, ROWS_PER_TILE)],
    )


@functools.cache
def _sc_seg_sum_kernel():
    return functools.partial(
        pl.kernel,
        out_type=jax.ShapeDtypeStruct((NC, N_PAD, D), jnp.float32),
        mesh=_mesh(),
        scratch_types=[
            pltpu.VMEM((2, GS, K), jnp.int32),
            pltpu.VMEM((GS, K), jnp.int32),
            pltpu.VMEM((NBUF, K, D), jnp.float32),
            pltpu.VMEM_SHARED((N_PAD, D), jnp.float32),
            [pltpu.SemaphoreType.DMA] * NBUF,
        ],
        name="sc_seg_sum",
    )(_sc_seg_sum_body)


def _sc_seg_sum_body(
    src_hbm, dst_hbm, g_hbm, out_hbm, src_sl, dst_sl, rows_v, acc_sh, gsem
):
    c = lax.axis_index("c")
    s = lax.axis_index("s")
    base = s * ROWS_PER_TILE
    n0, n1 = NCH_C
    nch = jnp.where(c == 0, n0, n1)
    nslab2 = nch // (2 * GS)
    chunk0 = jnp.where(c == 0, s * n0, NS * n0 + s * n1)

    _zero_rows3(rows_v)
    for r in range(ROWS_PER_TILE // K):
        pltpu.sync_copy(rows_v.at[0], acc_sh.at[pl.ds(base + r * K, K)])
    plsc.subcore_barrier()

    def gather(sl_par, loc, b):
        pltpu.async_copy(g_hbm.at[src_sl.at[sl_par, loc]], rows_v.at[b], gsem[b])

    def gather_wait(b):
        pltpu.make_async_copy(g_hbm.at[src_sl.at[0, 0]], rows_v.at[b], gsem[b]).wait()

    pltpu.sync_copy(src_hbm.at[pl.ds(chunk0, GS)], src_sl.at[0])
    for b in range(NBUF):
        gather(0, b, b)

    def slab_pair(g2, _):
        for par in range(2):
            g = 2 * g2 + par
            gbase = chunk0 + g * GS
            @pl.when(g + 1 < nch // GS)
            def _():
                pltpu.sync_copy(src_hbm.at[pl.ds(gbase + GS, GS)], src_sl.at[1 - par])

            pltpu.sync_copy(dst_hbm.at[pl.ds(gbase, GS)], dst_sl)
            for b8 in range(GS):
                b = b8 % NBUF
                gather_wait(b)
                pltpu.sync_copy(rows_v.at[b], acc_sh.at[dst_sl.at[b8]], add=True)
                jn = b8 + NBUF
                nxt_par = par if jn < GS else 1 - par

                @pl.when(g * GS + b8 + NBUF < nch)
                def _():
                    gather(nxt_par, jn % GS, b)

        return 0

    lax.fori_loop(0, nslab2, slab_pair, 0)
    plsc.subcore_barrier()

    pltpu.sync_copy(
        acc_sh.at[pl.ds(base, ROWS_PER_TILE)],
        out_hbm.at[c, pl.ds(base, ROWS_PER_TILE)],
    )


def _dinv_block(hist_blk):
    deg = hist_blk[0, :, :1] + hist_blk[1, :, :1] + 1.0
    return lax.rsqrt(deg)


def _tc_a_body(hist_ref, x_ref, w1_ref, g1_ref):
    dinv = _dinv_block(hist_ref[...])
    h = jnp.dot(x_ref[...], w1_ref[...], preferred_element_type=jnp.float32)
    g1_ref[...] = dinv * h


def _tc_b_body(hist_ref, agg_ref, g1_ref, b1_ref, w2_ref, g2_ref):
    dinv = _dinv_block(hist_ref[...])
    a = agg_ref[0] + agg_ref[1] + g1_ref[...]
    u = jnp.maximum(dinv * a + b1_ref[...], 0.0)
    g2_ref[...] = dinv * jnp.dot(u, w2_ref[...], preferred_element_type=jnp.float32)


def _tc_c_body(hist_ref, agg_ref, g2_ref, b2_ref, wp_ref, bp_ref, out_ref):
    dinv = _dinv_block(hist_ref[...])
    a = agg_ref[0] + agg_ref[1] + g2_ref[...]
    o = dinv * a + b2_ref[...]
    out_ref[...] = jnp.dot(o, wp_ref[...], preferred_element_type=jnp.float32) + bp_ref[...]


def _row_specs(rb, extra):
    hist = pl.BlockSpec((2, rb, D), lambda i: (0, i, 0))
    row = pl.BlockSpec((rb, D), lambda i: (i, 0))
    agg = pl.BlockSpec((2, rb, D), lambda i: (0, i, 0))
    full = pl.BlockSpec((1, D), lambda i: (0, 0))
    mat = pl.BlockSpec((D, D), lambda i: (0, 0))
    m = {"hist": hist, "row": row, "agg": agg, "full": full, "mat": mat}
    return [m[e] for e in extra]


def _tc_a(hist, x_pad, w1):
    rb = N_PAD // 8
    return pl.pallas_call(
        _tc_a_body,
        grid=(8,),
        in_specs=_row_specs(rb, ["hist", "row", "mat"]),
        out_specs=pl.BlockSpec((rb, D), lambda i: (i, 0)),
        out_shape=jax.ShapeDtypeStruct((N_PAD, D), jnp.float32),
    )(hist, x_pad, w1)


def _tc_b(hist, agg, g1, b1, w2):
    rb = N_PAD // 8
    return pl.pallas_call(
        _tc_b_body,
        grid=(8,),
        in_specs=_row_specs(rb, ["hist", "agg", "row", "full", "mat"]),
        out_specs=pl.BlockSpec((rb, D), lambda i: (i, 0)),
        out_shape=jax.ShapeDtypeStruct((N_PAD, D), jnp.float32),
    )(hist, agg, g1, b1, w2)


def _tc_c(hist, agg, g2, b2, wp, bp):
    rb = 1256
    return pl.pallas_call(
        _tc_c_body,
        grid=(8,),
        in_specs=_row_specs(rb, ["hist", "agg", "row", "full", "mat", "full"]),
        out_specs=pl.BlockSpec((rb, D), lambda i: (i, 0)),
        out_shape=jax.ShapeDtypeStruct((N_NODES, D), jnp.float32),
    )(hist, agg, g2, b2, wp, bp)


def kernel(x, edge_index, W1, b1, W2, b2, Wp, bp):
    src = edge_index[0].astype(jnp.int32)
    dst = edge_index[1].astype(jnp.int32)

    fill = jnp.full((E_PAD,), N_NODES, jnp.int32)
    srcf = fill.at[: src.shape[0]].set(src)
    dstf = fill.at[: dst.shape[0]].set(dst)
    dst3 = dstf.reshape(NW, NCH, K)
    src2 = srcf.reshape(CH_TOT, K)
    dst2 = dstf.reshape(CH_TOT, K)

    x_pad = jnp.zeros((N_PAD, D), jnp.float32).at[:N_NODES].set(x)
    b1r = b1.reshape(1, D)
    b2r = b2.reshape(1, D)
    bpr = bp.reshape(1, D)

    hist = _sc_deg_hist_kernel()(dst3)
    g1 = _tc_a(hist, x_pad, W1)
    agg1 = _sc_seg_sum_kernel()(src2, dst2, g1)
    g2 = _tc_b(hist, agg1, g1, b1r, W2)
    agg2 = _sc_seg_sum_kernel()(src2, dst2, g2)
    out = _tc_c(hist, agg2, g2, b2r, Wp, bpr)
    return out


# speedup vs baseline: 11.3807x; 1.1561x over previous
import functools

import jax
import jax.numpy as jnp
from jax import lax
from jax.experimental import pallas as pl
from jax.experimental.pallas import tpu as pltpu
from jax.experimental.pallas import tpu_sc as plsc

N_NODES = 10000
D = 128

NC = 2
NS = 16
NW = NC * NS

N_PAD = 10240
ROWS_PER_TILE = N_PAD // NS
K = 128
E_PAD_PER_W = 10240
NCH = E_PAD_PER_W // K
E_PAD = NW * E_PAD_PER_W
NBUF = 2
GS = 8
NCH_C = (144, 16)
CH_TOT = E_PAD // K

_mesh = lambda: plsc.VectorSubcoreMesh(
    core_axis_name="c", subcore_axis_name="s", num_cores=NC, num_subcores=NS
)


def _zero_rows(buf, nrows, width):
    z16 = jnp.zeros((16,), jnp.float32)

    def body(i, _):
        for j in range(width // 16):
            buf[i, pl.ds(j * 16, 16)] = z16
        return 0

    lax.fori_loop(0, nrows, body, 0)


def _zero_rows3(buf):
    z16 = jnp.zeros((16,), jnp.float32)

    def body(i, _):
        for j in range(D // 16):
            buf[0, i, pl.ds(j * 16, 16)] = z16
        return 0

    lax.fori_loop(0, K, body, 0)


def _fill_ones(buf, nrows, width):
    o16 = jnp.ones((16,), jnp.float32)

    def body(i, _):
        for j in range(width // 16):
            buf[i, pl.ds(j * 16, 16)] = o16
        return 0

    lax.fori_loop(0, nrows, body, 0)


@functools.cache
def _sc_deg_hist_kernel():
    return functools.partial(
        pl.kernel,
        out_type=jax.ShapeDtypeStruct((NC, N_PAD, D), jnp.float32),
        mesh=_mesh(),
        scratch_types=[
            pltpu.VMEM((NCH, K), jnp.int32),
            pltpu.VMEM((K, D), jnp.float32),
            pltpu.VMEM_SHARED((N_PAD, D), jnp.float32),
        ],
        name="sc_deg_hist",
    )(_sc_deg_hist_body)


def _sc_deg_hist_body(dst_hbm, out_hbm, dst_v, buf_v, acc_sh):
    c = lax.axis_index("c")
    s = lax.axis_index("s")
    wid = c * NS + s
    base = s * ROWS_PER_TILE

    _zero_rows(buf_v, K, D)
    for r in range(ROWS_PER_TILE // K):
        pltpu.sync_copy(buf_v, acc_sh.at[pl.ds(base + r * K, K)])
    rem = ROWS_PER_TILE % K
    if rem:
        pltpu.sync_copy(
            buf_v.at[pl.ds(0, rem)],
            acc_sh.at[pl.ds(base + (ROWS_PER_TILE // K) * K, rem)],
        )
    plsc.subcore_barrier()

    _fill_ones(buf_v, K, D)
    pltpu.sync_copy(dst_hbm.at[wid], dst_v)

    def chunk(j, _):
        pltpu.sync_copy(buf_v, acc_sh.at[dst_v.at[j]], add=True)
        return 0

    lax.fori_loop(0, NCH, chunk, 0)
    plsc.subcore_barrier()

    pltpu.sync_copy(
        acc_sh.at[pl.ds(base, ROWS_PER_TILE)],
        out_hbm.at[c, pl.ds(base, ROWS_PER_TILE)],
    )


@functools.cache
def _sc_seg_sum_kernel():
    return functools.partial(
        pl.kernel,
        out_type=jax.ShapeDtypeStruct((NC, N_PAD, D), jnp.float32),
        mesh=_mesh(),
        scratch_types=[
            pltpu.VMEM((2, GS, K), jnp.int32),
            pltpu.VMEM((GS, K), jnp.int32),
            pltpu.VMEM((NBUF, K, D), jnp.float32),
            pltpu.VMEM_SHARED((N_PAD, D), jnp.float32),
            [pltpu.SemaphoreType.DMA] * NBUF,
        ],
        name="sc_seg_sum",
    )(_sc_seg_sum_body)


def _sc_seg_sum_body(
    src_hbm, dst_hbm, g_hbm, out_hbm, src_sl, dst_sl, rows_v, acc_sh, gsem
):
    c = lax.axis_index("c")
    s = lax.axis_index("s")
    base = s * ROWS_PER_TILE
    n0, n1 = NCH_C
    nch = jnp.where(c == 0, n0, n1)
    nslab2 = nch // (2 * GS)
    chunk0 = jnp.where(c == 0, s * n0, NS * n0 + s * n1)

    _zero_rows3(rows_v)
    for r in range(ROWS_PER_TILE // K):
        pltpu.sync_copy(rows_v.at[0], acc_sh.at[pl.ds(base + r * K, K)])
    plsc.subcore_barrier()

    def gather(sl_par, loc, b):
        pltpu.async_copy(g_hbm.at[src_sl.at[sl_par, loc]], rows_v.at[b], gsem[b])

    def gather_wait(b):
        pltpu.make_async_copy(g_hbm.at[src_sl.at[0, 0]], rows_v.at[b], gsem[b]).wait()

    pltpu.sync_copy(src_hbm.at[pl.ds(chunk0, GS)], src_sl.at[0])
    for b in range(NBUF):
        gather(0, b, b)

    def slab_pair(g2, _):
        for par in range(2):
            g = 2 * g2 + par
            gbase = chunk0 + g * GS
            @pl.when(g + 1 < nch // GS)
            def _():
                pltpu.sync_copy(src_hbm.at[pl.ds(gbase + GS, GS)], src_sl.at[1 - par])

            pltpu.sync_copy(dst_hbm.at[pl.ds(gbase, GS)], dst_sl)
            for b8 in range(GS):
                b = b8 % NBUF
                gather_wait(b)
                pltpu.sync_copy(rows_v.at[b], acc_sh.at[dst_sl.at[b8]], add=True)
                jn = b8 + NBUF
                nxt_par = par if jn < GS else 1 - par

                @pl.when(g * GS + b8 + NBUF < nch)
                def _():
                    gather(nxt_par, jn % GS, b)

        return 0

    lax.fori_loop(0, nslab2, slab_pair, 0)
    plsc.subcore_barrier()

    pltpu.sync_copy(
        acc_sh.at[pl.ds(base, ROWS_PER_TILE)],
        out_hbm.at[c, pl.ds(base, ROWS_PER_TILE)],
    )


def _dinv_block(hist_blk):
    deg = hist_blk[0, :, :1] + hist_blk[1, :, :1] + 1.0
    return lax.rsqrt(deg)


def _tc_a_body(hist_ref, x_ref, w1_ref, g1_ref):
    dinv = _dinv_block(hist_ref[...])
    h = jnp.dot(x_ref[...], w1_ref[...], preferred_element_type=jnp.float32)
    g1_ref[...] = dinv * h


def _tc_b_body(hist_ref, agg_ref, g1_ref, b1_ref, w2_ref, g2_ref):
    dinv = _dinv_block(hist_ref[...])
    a = agg_ref[0] + agg_ref[1] + g1_ref[...]
    u = jnp.maximum(dinv * a + b1_ref[...], 0.0)
    g2_ref[...] = dinv * jnp.dot(u, w2_ref[...], preferred_element_type=jnp.float32)


def _tc_c_body(hist_ref, agg_ref, g2_ref, b2_ref, wp_ref, bp_ref, out_ref):
    dinv = _dinv_block(hist_ref[...])
    a = agg_ref[0] + agg_ref[1] + g2_ref[...]
    o = dinv * a + b2_ref[...]
    out_ref[...] = jnp.dot(o, wp_ref[...], preferred_element_type=jnp.float32) + bp_ref[...]


def _row_specs(rb, extra):
    hist = pl.BlockSpec((2, rb, D), lambda i: (0, i, 0))
    row = pl.BlockSpec((rb, D), lambda i: (i, 0))
    agg = pl.BlockSpec((2, rb, D), lambda i: (0, i, 0))
    full = pl.BlockSpec((1, D), lambda i: (0, 0))
    mat = pl.BlockSpec((D, D), lambda i: (0, 0))
    m = {"hist": hist, "row": row, "agg": agg, "full": full, "mat": mat}
    return [m[e] for e in extra]


def _tc_a(hist, x_pad, w1):
    rb = N_PAD // 8
    return pl.pallas_call(
        _tc_a_body,
        grid=(8,),
        in_specs=_row_specs(rb, ["hist", "row", "mat"]),
        out_specs=pl.BlockSpec((rb, D), lambda i: (i, 0)),
        out_shape=jax.ShapeDtypeStruct((N_PAD, D), jnp.float32),
    )(hist, x_pad, w1)


def _tc_b(hist, agg, g1, b1, w2):
    rb = N_PAD // 8
    return pl.pallas_call(
        _tc_b_body,
        grid=(8,),
        in_specs=_row_specs(rb, ["hist", "agg", "row", "full", "mat"]),
        out_specs=pl.BlockSpec((rb, D), lambda i: (i, 0)),
        out_shape=jax.ShapeDtypeStruct((N_PAD, D), jnp.float32),
    )(hist, agg, g1, b1, w2)


def _tc_c(hist, agg, g2, b2, wp, bp):
    rb = N_PAD // 8
    return pl.pallas_call(
        _tc_c_body,
        grid=(8,),
        in_specs=_row_specs(rb, ["hist", "agg", "row", "full", "mat", "full"]),
        out_specs=pl.BlockSpec((rb, D), lambda i: (i, 0)),
        out_shape=jax.ShapeDtypeStruct((N_NODES, D), jnp.float32),
    )(hist, agg, g2, b2, wp, bp)


def kernel(x, edge_index, W1, b1, W2, b2, Wp, bp):
    src = edge_index[0].astype(jnp.int32)
    dst = edge_index[1].astype(jnp.int32)

    fill = jnp.full((E_PAD,), N_NODES, jnp.int32)
    srcf = fill.at[: src.shape[0]].set(src)
    dstf = fill.at[: dst.shape[0]].set(dst)
    dst3 = dstf.reshape(NW, NCH, K)
    src2 = srcf.reshape(CH_TOT, K)
    dst2 = dstf.reshape(CH_TOT, K)

    x_pad = jnp.zeros((N_PAD, D), jnp.float32).at[:N_NODES].set(x)
    b1r = b1.reshape(1, D)
    b2r = b2.reshape(1, D)
    bpr = bp.reshape(1, D)

    hist = _sc_deg_hist_kernel()(dst3)
    g1 = _tc_a(hist, x_pad, W1)
    agg1 = _sc_seg_sum_kernel()(src2, dst2, g1)
    g2 = _tc_b(hist, agg1, g1, b1r, W2)
    agg2 = _sc_seg_sum_kernel()(src2, dst2, g2)
    out = _tc_c(hist, agg2, g2, b2r, Wp, bpr)
    return out
